# baseline (speedup 1.0000x reference)
import functools

import jax
import jax.numpy as jnp
from jax import lax
from jax.experimental import pallas as pl
from jax.experimental.pallas import tpu as pltpu
from jax.experimental.pallas import tpu_sc as plsc

NUM_CORES = 2
NUM_SUBCORES = 16
NUM_WORKERS = NUM_CORES * NUM_SUBCORES
LANES = 16

NUM_USERS = 1000
NUM_ITEMS = 1000000
BATCH = 16384
EMBED_DIM = 64
B_PER_W = BATCH // NUM_WORKERS
N_GROUPS = B_PER_W // LANES
SPLIT = 614544


def _body(items_hbm, uoff_hbm, utf_hbm, it_hbm, itb_hbm, out_hbm,
          items_v, uoff_v, u_tab, i_rows, out_v, sem):
    c = lax.axis_index("c")
    s = lax.axis_index("s")
    wid = s * NUM_CORES + c
    base = wid * B_PER_W

    pltpu.sync_copy(items_hbm.at[pl.ds(base, B_PER_W)], items_v)
    pltpu.sync_copy(uoff_hbm.at[pl.ds(base, B_PER_W)], uoff_v)
    pltpu.sync_copy(utf_hbm, u_tab)

    def issue(g, _):
        iv = items_v[pl.ds(g * LANES, LANES)]
        for k in range(LANES):
            r = g * LANES + k
            row = iv[k]

            @pl.when(row < SPLIT)
            def _():
                pltpu.async_copy(it_hbm.at[row >> 3, pl.ds(row & 7, 1), :],
                                 i_rows.at[pl.ds(r, 1), :], sem)

            @pl.when(row >= SPLIT)
            def _():
                pltpu.async_copy(itb_hbm.at[pl.ds(row - SPLIT, 1), :],
                                 i_rows.at[pl.ds(r, 1), :], sem)
        return 0

    lax.fori_loop(0, N_GROUPS, issue, 0)

    for j in range(B_PER_W // 8):
        pltpu.make_async_copy(it_hbm.at[0, :, :],
                              i_rows.at[pl.ds(j * 8, 8), :], sem).wait()

    last_lane = lax.iota(jnp.int32, LANES) == (LANES - 1)

    def pair_block(g, _):
        uo = uoff_v[pl.ds(g * LANES, LANES)]
        for k in range(LANES):
            r = g * LANES + k
            ub = uo[k]
            acc = (u_tab[pl.ds(ub, LANES)]
                   * i_rows[r, pl.ds(0, LANES)])
            for q in range(1, EMBED_DIM // LANES):
                acc = acc + (u_tab[pl.ds(ub + q * LANES, LANES)]
                             * i_rows[r, pl.ds(q * LANES, LANES)])
            csum = plsc.cumsum(acc)
            pos = jnp.zeros((LANES,), jnp.int32) + r
            plsc.store_scatter(out_v, [pos], csum, mask=last_lane)
        return 0

    lax.fori_loop(0, N_GROUPS, pair_block, 0)

    pltpu.sync_copy(out_v, out_hbm.at[pl.ds(base, B_PER_W)])


def kernel(users, items, user_table, item_table):
    it3 = item_table[:SPLIT].reshape(SPLIT // 8, 8, EMBED_DIM)
    itb = item_table[SPLIT:]
    utf = user_table.reshape(NUM_USERS * EMBED_DIM)
    uoff = users * EMBED_DIM
    mesh = plsc.VectorSubcoreMesh(core_axis_name="c", subcore_axis_name="s")
    run = pl.kernel(
        _body,
        out_type=jax.ShapeDtypeStruct((BATCH,), jnp.float32),
        mesh=mesh,
        scratch_types=[
            pltpu.VMEM((B_PER_W,), jnp.int32),
            pltpu.VMEM((B_PER_W,), jnp.int32),
            pltpu.VMEM((NUM_USERS * EMBED_DIM,), jnp.float32),
            pltpu.VMEM((B_PER_W, EMBED_DIM), jnp.float32),
            pltpu.VMEM((B_PER_W,), jnp.float32),
            pltpu.SemaphoreType.DMA,
        ],
        compiler_params=pltpu.CompilerParams(needs_layout_passes=False),
    )
    return run(items, uoff, utf, it3, itb)


# speedup vs baseline: 1.6882x; 1.6882x over previous
import functools

import jax
import jax.numpy as jnp
from jax import lax
from jax.experimental import pallas as pl
from jax.experimental.pallas import tpu as pltpu
from jax.experimental.pallas import tpu_sc as plsc

NUM_CORES = 2
NUM_SUBCORES = 16
NUM_WORKERS = NUM_CORES * NUM_SUBCORES
LANES = 16

NUM_USERS = 1000
NUM_ITEMS = 1000000
BATCH = 16384
EMBED_DIM = 64
B_PER_W = BATCH // NUM_WORKERS
N_GROUPS = B_PER_W // LANES


def _body(items_hbm, uoff_hbm, utf_hbm, it_hbm, out_hbm,
          items_v, uoff_v, u_tab, i_rows, out_v, sem_a, sem_b, sem_u):
    c = lax.axis_index("c")
    s = lax.axis_index("s")
    wid = s * NUM_CORES + c
    base = wid * B_PER_W

    pltpu.sync_copy(items_hbm.at[pl.ds(base, B_PER_W)], items_v)
    pltpu.sync_copy(uoff_hbm.at[pl.ds(base, B_PER_W)], uoff_v)
    u_cp = pltpu.async_copy(utf_hbm, u_tab, sem_u)

    def issue(g, sem):
        def body(gg, _, sem=sem):
            iv = items_v[pl.ds(gg * LANES, LANES)]
            for k in range(LANES):
                r = gg * LANES + k
                pltpu.async_copy(it_hbm.at[iv[k] >> 3, pl.ds(iv[k] & 7, 1), :],
                                 i_rows.at[pl.ds(r, 1), :], sem)
            return 0
        return body

    lax.fori_loop(0, N_GROUPS // 2, issue(0, sem_a), 0)
    lax.fori_loop(N_GROUPS // 2, N_GROUPS, issue(0, sem_b), 0)
    u_cp.wait()

    last_lane = lax.iota(jnp.int32, LANES) == (LANES - 1)

    def pair_block(g, _):
        uo = uoff_v[pl.ds(g * LANES, LANES)]
        for k in range(LANES):
            r = g * LANES + k
            ub = uo[k]
            acc = (u_tab[pl.ds(ub, LANES)]
                   * i_rows[r, pl.ds(0, LANES)])
            for q in range(1, EMBED_DIM // LANES):
                acc = acc + (u_tab[pl.ds(ub + q * LANES, LANES)]
                             * i_rows[r, pl.ds(q * LANES, LANES)])
            csum = plsc.cumsum(acc)
            pos = jnp.zeros((LANES,), jnp.int32) + r
            plsc.store_scatter(out_v, [pos], csum, mask=last_lane)
        return 0

    for h, sem in ((0, sem_a), (1, sem_b)):
        for j in range(B_PER_W // 16):
            pltpu.make_async_copy(
                it_hbm.at[0, :, :],
                i_rows.at[pl.ds(h * (B_PER_W // 2) + j * 8, 8), :], sem).wait()
        lax.fori_loop(h * N_GROUPS // 2, (h + 1) * N_GROUPS // 2, pair_block, 0)

    pltpu.sync_copy(out_v, out_hbm.at[pl.ds(base, B_PER_W)])


def kernel(users, items, user_table, item_table):
    it3 = item_table.reshape(NUM_ITEMS // 8, 8, EMBED_DIM)
    utf = user_table.reshape(NUM_USERS * EMBED_DIM)
    uoff = users * EMBED_DIM
    mesh = plsc.VectorSubcoreMesh(core_axis_name="c", subcore_axis_name="s")
    run = pl.kernel(
        _body,
        out_type=jax.ShapeDtypeStruct((BATCH,), jnp.float32),
        mesh=mesh,
        scratch_types=[
            pltpu.VMEM((B_PER_W,), jnp.int32),
            pltpu.VMEM((B_PER_W,), jnp.int32),
            pltpu.VMEM((NUM_USERS * EMBED_DIM,), jnp.float32),
            pltpu.VMEM((B_PER_W, EMBED_DIM), jnp.float32),
            pltpu.VMEM((B_PER_W,), jnp.float32),
            pltpu.SemaphoreType.DMA,
            pltpu.SemaphoreType.DMA,
            pltpu.SemaphoreType.DMA,
        ],
        compiler_params=pltpu.CompilerParams(needs_layout_passes=False),
    )
    return run(items, uoff, utf, it3)
